# baseline (speedup 1.0000x reference)
import functools

import jax
import jax.numpy as jnp
from jax import lax
from jax.experimental import pallas as pl
from jax.experimental.pallas import tpu as pltpu
from jax.experimental.pallas import tpu_sc as plsc

_NC = 2
_NS = 16
_NW = _NC * _NS
_CH = 128
_NSEG = 4
_LANES = 16


def _sc_gather_scatter(n_nodes, n_hid, e_pad, hrel, gidx, dst, nrm):
    epw = e_pad // _NW
    nchunk = epw // _CH
    n_pad = ((n_nodes + _NS * 8 - 1) // (_NS * 8)) * (_NS * 8)
    rpt = n_pad // _NS
    nvec = n_hid // _LANES

    mesh = plsc.VectorSubcoreMesh(core_axis_name="c", subcore_axis_name="s",
                                  num_cores=_NC, num_subcores=_NS)

    nseg = _NSEG
    cps = nchunk // nseg

    @functools.partial(
        pl.kernel,
        out_type=jax.ShapeDtypeStruct((_NC * n_pad, n_hid), jnp.float32),
        mesh=mesh,
        scratch_types=[
            pltpu.VMEM((cps, _CH), jnp.int32),
            pltpu.VMEM((cps, _CH), jnp.int32),
            pltpu.VMEM((cps * _CH,), jnp.float32),
            pltpu.VMEM((_CH, n_hid), jnp.float32),
            pltpu.VMEM((_CH, n_hid), jnp.float32),
            pltpu.VMEM_SHARED((n_pad, n_hid), jnp.float32),
            pltpu.SemaphoreType.DMA,
            pltpu.SemaphoreType.DMA,
            pltpu.SemaphoreType.DMA,
            pltpu.SemaphoreType.DMA,
        ],
    )
    def k(hrel_hbm, gidx_hbm, dst_hbm, nrm_hbm, out_hbm,
          idx_v, dst_v, nrm_v, rows0, rows1, agg_sh, sem0, sem1, ssem0, ssem1):
        cid = lax.axis_index("c")
        sid = lax.axis_index("s")
        wid = cid * _NS + sid

        zeros16 = jnp.zeros((_LANES,), jnp.float32)

        def zero_row(i, carry):
            for j in range(nvec):
                rows0[i, pl.ds(j * _LANES, _LANES)] = zeros16
            return carry

        lax.fori_loop(0, _CH, zero_row, 0)
        row0 = sid * rpt
        done = 0
        while done < rpt:
            sz = min(_CH, rpt - done)
            pltpu.sync_copy(rows0.at[pl.ds(0, sz)],
                            agg_sh.at[pl.ds(row0 + done, sz)])
            done += sz
        plsc.subcore_barrier()

        def scale(t, buf):
            def group(g, c2):
                norms = nrm_v[pl.ds(t * _CH + g * _LANES, _LANES)]
                e0 = g * _LANES
                for lane in range(_LANES):
                    s = norms[lane]
                    for j in range(nvec):
                        sl = pl.ds(j * _LANES, _LANES)
                        buf[e0 + lane, sl] = buf[e0 + lane, sl] * s
                return c2

            lax.fori_loop(0, _CH // _LANES, group, 0)

        def seg(s2, carry):
            pltpu.sync_copy(gidx_hbm.at[wid, s2], idx_v)
            pltpu.sync_copy(dst_hbm.at[wid, s2], dst_v)
            pltpu.sync_copy(nrm_hbm.at[wid, s2], nrm_v)
            pltpu.async_copy(hrel_hbm.at[idx_v.at[0]], rows0, sem0)

            def pair(p, c2):
                t0 = 2 * p
                t1 = t0 + 1

                @pl.when(p > 0)
                def _():
                    pltpu.make_async_copy(rows1, agg_sh.at[dst_v.at[t1 - 2]],
                                          ssem1).wait()

                pltpu.async_copy(hrel_hbm.at[idx_v.at[t1]], rows1, sem1)
                pltpu.make_async_copy(hrel_hbm.at[idx_v.at[t0]], rows0,
                                      sem0).wait()
                scale(t0, rows0)
                pltpu.async_copy(rows0, agg_sh.at[dst_v.at[t0]], ssem0,
                                 add=True)
                pltpu.make_async_copy(hrel_hbm.at[idx_v.at[t1]], rows1,
                                      sem1).wait()
                scale(t1, rows1)

                @pl.when(t1 + 1 < cps)
                def _():
                    pltpu.make_async_copy(rows0, agg_sh.at[dst_v.at[t0]],
                                          ssem0).wait()
                    pltpu.async_copy(hrel_hbm.at[idx_v.at[t1 + 1]], rows0, sem0)

                pltpu.async_copy(rows1, agg_sh.at[dst_v.at[t1]], ssem1,
                                 add=True)
                return c2

            lax.fori_loop(0, cps // 2, pair, 0)
            pltpu.make_async_copy(rows0, agg_sh.at[dst_v.at[cps - 2]],
                                  ssem0).wait()
            pltpu.make_async_copy(rows1, agg_sh.at[dst_v.at[cps - 1]],
                                  ssem1).wait()
            return carry

        lax.fori_loop(0, nseg, seg, 0)

        plsc.subcore_barrier()
        done = 0
        while done < rpt:
            sz = min(_CH, rpt - done)
            pltpu.sync_copy(agg_sh.at[pl.ds(row0 + done, sz)],
                            out_hbm.at[pl.ds(cid * n_pad + row0 + done, sz)])
            done += sz

    out = k(hrel,
            gidx.reshape(_NW, nseg, cps, _CH),
            dst.reshape(_NW, nseg, cps, _CH),
            nrm.reshape(_NW, nseg, cps * _CH))
    return out.reshape(_NC, n_pad, n_hid)[:, :n_nodes, :]


def _matmul(x, ws):
    n, d = x.shape
    r1, _, h = ws.shape
    bn = 2000
    nb = n // bn

    def body(x_ref, w_ref, o_ref):
        o_ref[0] = jnp.dot(x_ref[...], w_ref[0],
                           preferred_element_type=jnp.float32)

    return pl.pallas_call(
        body,
        grid=(nb, r1),
        in_specs=[pl.BlockSpec((bn, d), lambda i, r: (i, 0)),
                  pl.BlockSpec((1, d, h), lambda i, r: (r, 0, 0))],
        out_specs=pl.BlockSpec((1, bn, h), lambda i, r: (r, i, 0)),
        out_shape=jax.ShapeDtypeStruct((r1, n, h), jnp.float32),
    )(x, ws)


def _combine_matmul(parts, hall, b, ws):
    n, hdim = parts.shape[1], parts.shape[2]
    r1, _, out_dim = ws.shape
    loop_row = hall.shape[0] - 1
    bn = 2000
    nb = n // bn

    def body(p_ref, lt_ref, b_ref, w_ref, o_ref):
        hblk = p_ref[0] + p_ref[1] + lt_ref[0] + b_ref[0]
        hblk = jnp.maximum(hblk, 0.0)
        o_ref[0] = jnp.dot(hblk, w_ref[0], preferred_element_type=jnp.float32)

    return pl.pallas_call(
        body,
        grid=(nb, r1),
        in_specs=[pl.BlockSpec((2, bn, hdim), lambda i, r: (0, i, 0)),
                  pl.BlockSpec((1, bn, hdim), lambda i, r: (loop_row, i, 0)),
                  pl.BlockSpec((1, hdim), lambda i, r: (0, 0)),
                  pl.BlockSpec((1, hdim, out_dim), lambda i, r: (r, 0, 0))],
        out_specs=pl.BlockSpec((1, bn, out_dim), lambda i, r: (r, i, 0)),
        out_shape=jax.ShapeDtypeStruct((r1, n, out_dim), jnp.float32),
    )(parts, hall, b.reshape(1, -1), ws)


def _final_sum(parts, hall, b):
    n, d = parts.shape[1], parts.shape[2]
    loop_row = hall.shape[0] - 1
    bn = 2000
    nb = n // bn

    def body(p_ref, lt_ref, b_ref, o_ref):
        o_ref[...] = p_ref[0] + p_ref[1] + lt_ref[0] + b_ref[0]

    return pl.pallas_call(
        body,
        grid=(nb,),
        in_specs=[pl.BlockSpec((2, bn, d), lambda i: (0, i, 0)),
                  pl.BlockSpec((1, bn, d), lambda i: (loop_row, i, 0)),
                  pl.BlockSpec((1, d), lambda i: (0, 0))],
        out_specs=pl.BlockSpec((bn, d), lambda i: (i, 0)),
        out_shape=jax.ShapeDtypeStruct((n, d), jnp.float32),
    )(parts, hall, b.reshape(1, -1))


def kernel(features, edge_index, etypes, norm, W1, loop1, b1, W2, loop2, b2):
    n, _ = features.shape
    e = etypes.shape[0]
    src = edge_index[0].astype(jnp.int32)
    dstv = edge_index[1].astype(jnp.int32)
    et = etypes.astype(jnp.int32)
    gidx = et * n + src
    nrm = norm[:, 0]

    granule = _NW * _CH * _NSEG
    e_pad = ((e + granule - 1) // granule) * granule
    pad = e_pad - e
    if pad:
        spread = jnp.arange(pad, dtype=jnp.int32) % n
        gidx = jnp.concatenate([gidx, spread])
        dstv = jnp.concatenate([dstv, spread])
        nrm = jnp.concatenate([nrm, jnp.zeros((pad,), jnp.float32)])

    ws1 = jnp.concatenate([W1, loop1[None]], axis=0)
    ws2 = jnp.concatenate([W2, loop2[None]], axis=0)

    hall1 = _matmul(features, ws1)
    p1 = _sc_gather_scatter(n, hall1.shape[2], e_pad,
                            hall1.reshape(-1, hall1.shape[2]), gidx, dstv, nrm)
    hall2 = _combine_matmul(p1, hall1, b1, ws2)
    p2 = _sc_gather_scatter(n, hall2.shape[2], e_pad,
                            hall2.reshape(-1, hall2.shape[2]), gidx, dstv, nrm)
    return _final_sum(p2, hall2, b2)


# speedup vs baseline: 1.0661x; 1.0661x over previous
import functools

import jax
import jax.numpy as jnp
from jax import lax
from jax.experimental import pallas as pl
from jax.experimental.pallas import tpu as pltpu
from jax.experimental.pallas import tpu_sc as plsc

_NC = 2
_NS = 16
_NW = _NC * _NS
_CH = 128
_NSEG = 4
_LANES = 16


def _sc_gather_scatter(n_nodes, n_hid, e_pad, hrel, gidx, dst, nrm):
    epw = e_pad // _NW
    nchunk = epw // _CH
    n_pad = ((n_nodes + _NS * 8 - 1) // (_NS * 8)) * (_NS * 8)
    rpt = n_pad // _NS
    nvec = n_hid // _LANES

    mesh = plsc.VectorSubcoreMesh(core_axis_name="c", subcore_axis_name="s",
                                  num_cores=_NC, num_subcores=_NS)

    nseg = _NSEG
    cps = nchunk // nseg

    @functools.partial(
        pl.kernel,
        out_type=jax.ShapeDtypeStruct((_NC * n_pad, n_hid), jnp.float32),
        mesh=mesh,
        scratch_types=[
            pltpu.VMEM((cps, _CH), jnp.int32),
            pltpu.VMEM((cps, _CH), jnp.int32),
            pltpu.VMEM((cps * _CH,), jnp.float32),
            pltpu.VMEM((_CH, n_hid), jnp.float32),
            pltpu.VMEM((_CH, n_hid), jnp.float32),
            pltpu.VMEM_SHARED((n_pad, n_hid), jnp.float32),
            pltpu.SemaphoreType.DMA,
            pltpu.SemaphoreType.DMA,
        ],
    )
    def k(hrel_hbm, gidx_hbm, dst_hbm, nrm_hbm, out_hbm,
          idx_v, dst_v, nrm_v, rows0, rows1, agg_sh, sem0, sem1):
        cid = lax.axis_index("c")
        sid = lax.axis_index("s")
        wid = cid * _NS + sid

        zeros16 = jnp.zeros((_LANES,), jnp.float32)

        def zero_row(i, carry):
            for j in range(nvec):
                rows0[i, pl.ds(j * _LANES, _LANES)] = zeros16
            return carry

        lax.fori_loop(0, _CH, zero_row, 0)
        row0 = sid * rpt
        done = 0
        while done < rpt:
            sz = min(_CH, rpt - done)
            pltpu.sync_copy(rows0.at[pl.ds(0, sz)],
                            agg_sh.at[pl.ds(row0 + done, sz)])
            done += sz
        plsc.subcore_barrier()

        def scale(t, buf):
            def group(g, c2):
                norms = nrm_v[pl.ds(t * _CH + g * _LANES, _LANES)]
                e0 = g * _LANES
                for lane in range(_LANES):
                    s = norms[lane]
                    for j in range(nvec):
                        sl = pl.ds(j * _LANES, _LANES)
                        buf[e0 + lane, sl] = buf[e0 + lane, sl] * s
                return c2

            lax.fori_loop(0, _CH // _LANES, group, 0)

        def seg(s2, carry):
            pltpu.sync_copy(gidx_hbm.at[wid, s2], idx_v)
            pltpu.sync_copy(dst_hbm.at[wid, s2], dst_v)
            pltpu.sync_copy(nrm_hbm.at[wid, s2], nrm_v)
            pltpu.async_copy(hrel_hbm.at[idx_v.at[0]], rows0, sem0)

            def pair(p, c2):
                t0 = 2 * p
                t1 = t0 + 1
                pltpu.async_copy(hrel_hbm.at[idx_v.at[t1]], rows1, sem1)
                pltpu.make_async_copy(hrel_hbm.at[idx_v.at[t0]], rows0,
                                      sem0).wait()
                scale(t0, rows0)
                pltpu.sync_copy(rows0, agg_sh.at[dst_v.at[t0]], add=True)

                @pl.when(t1 + 1 < cps)
                def _():
                    pltpu.async_copy(hrel_hbm.at[idx_v.at[t1 + 1]], rows0, sem0)

                pltpu.make_async_copy(hrel_hbm.at[idx_v.at[t1]], rows1,
                                      sem1).wait()
                scale(t1, rows1)
                pltpu.sync_copy(rows1, agg_sh.at[dst_v.at[t1]], add=True)
                return c2

            lax.fori_loop(0, cps // 2, pair, 0)
            return carry

        lax.fori_loop(0, nseg, seg, 0)

        plsc.subcore_barrier()
        done = 0
        while done < rpt:
            sz = min(_CH, rpt - done)
            pltpu.sync_copy(agg_sh.at[pl.ds(row0 + done, sz)],
                            out_hbm.at[pl.ds(cid * n_pad + row0 + done, sz)])
            done += sz

    out = k(hrel,
            gidx.reshape(_NW, nseg, cps, _CH),
            dst.reshape(_NW, nseg, cps, _CH),
            nrm.reshape(_NW, nseg, cps * _CH))
    return out.reshape(_NC, n_pad, n_hid)


def _rel_matmul(x, ws):
    n, d = x.shape
    r1, _, h = ws.shape
    bn = 2000
    nb = n // bn

    def body(x_ref, w_ref, o_ref):
        o_ref[...] = jnp.dot(x_ref[...], w_ref[0],
                             preferred_element_type=jnp.float32)

    return pl.pallas_call(
        body,
        grid=(nb, r1),
        in_specs=[pl.BlockSpec((bn, d), lambda i, r: (i, 0)),
                  pl.BlockSpec((1, d, h), lambda i, r: (r, 0, 0))],
        out_specs=pl.BlockSpec((bn, h), lambda i, r: (r * nb + i, 0)),
        out_shape=jax.ShapeDtypeStruct((r1 * n, h), jnp.float32),
    )(x, ws)


def _combine_matmul(parts, x, wloop, b, ws):
    n, d = x.shape
    r1, _, h = ws.shape
    bn = 2000
    nb = n // bn

    def body(p_ref, x_ref, wl_ref, b_ref, w_ref, o2_ref, h1_ref, acc):
        r = pl.program_id(1)

        @pl.when(r == 0)
        def _():
            hblk = (p_ref[0] + p_ref[1] + b_ref[0]
                    + jnp.dot(x_ref[...], wl_ref[...],
                              preferred_element_type=jnp.float32))
            hblk = jnp.maximum(hblk, 0.0)
            acc[...] = hblk
            h1_ref[...] = hblk

        @pl.when(r > 0)
        def _():
            o2_ref[...] = jnp.dot(acc[...], w_ref[0],
                                  preferred_element_type=jnp.float32)

    hrel2, h1 = pl.pallas_call(
        body,
        grid=(nb, r1 + 1),
        in_specs=[pl.BlockSpec((2, bn, d), lambda i, r: (0, i, 0)),
                  pl.BlockSpec((bn, d), lambda i, r: (i, 0)),
                  pl.BlockSpec((d, d), lambda i, r: (0, 0)),
                  pl.BlockSpec((1, d), lambda i, r: (0, 0)),
                  pl.BlockSpec((1, d, h), lambda i, r: (jnp.maximum(r - 1, 0),
                                                        0, 0))],
        out_specs=[
            pl.BlockSpec((bn, h),
                         lambda i, r: (jnp.maximum(r - 1, 0) * nb + i, 0)),
            pl.BlockSpec((bn, d), lambda i, r: (i, 0)),
        ],
        out_shape=[jax.ShapeDtypeStruct((r1 * n, h), jnp.float32),
                   jax.ShapeDtypeStruct((n, d), jnp.float32)],
        scratch_shapes=[pltpu.VMEM((bn, d), jnp.float32)],
    )(parts, x, wloop, b.reshape(1, -1), ws)
    return hrel2, h1


def _final_sum(parts, h1, wloop, b):
    n, d = h1.shape
    bn = 2000
    nb = n // bn

    def body(p_ref, h_ref, wl_ref, b_ref, o_ref):
        o_ref[...] = (p_ref[0] + p_ref[1] + b_ref[0]
                      + jnp.dot(h_ref[...], wl_ref[...],
                                preferred_element_type=jnp.float32))

    return pl.pallas_call(
        body,
        grid=(nb,),
        in_specs=[pl.BlockSpec((2, bn, d), lambda i: (0, i, 0)),
                  pl.BlockSpec((bn, d), lambda i: (i, 0)),
                  pl.BlockSpec((d, d), lambda i: (0, 0)),
                  pl.BlockSpec((1, d), lambda i: (0, 0))],
        out_specs=pl.BlockSpec((bn, d), lambda i: (i, 0)),
        out_shape=jax.ShapeDtypeStruct((n, d), jnp.float32),
    )(parts, h1, wloop, b.reshape(1, -1))


def kernel(features, edge_index, etypes, norm, W1, loop1, b1, W2, loop2, b2):
    n, _ = features.shape
    e = etypes.shape[0]
    src = edge_index[0].astype(jnp.int32)
    dstv = edge_index[1].astype(jnp.int32)
    et = etypes.astype(jnp.int32)
    gidx = et * n + src
    nrm = norm[:, 0]

    granule = _NW * _CH * _NSEG
    e_pad = ((e + granule - 1) // granule) * granule
    pad = e_pad - e
    if pad:
        spread = jnp.arange(pad, dtype=jnp.int32) % n
        gidx = jnp.concatenate([gidx, spread])
        dstv = jnp.concatenate([dstv, spread])
        nrm = jnp.concatenate([nrm, jnp.zeros((pad,), jnp.float32)])

    hrel1 = _rel_matmul(features, W1)
    p1 = _sc_gather_scatter(n, W1.shape[2], e_pad, hrel1, gidx, dstv, nrm)
    hrel2, h1 = _combine_matmul(p1, features, loop1, b1, W2)
    p2 = _sc_gather_scatter(n, W2.shape[2], e_pad, hrel2, gidx, dstv, nrm)
    return _final_sum(p2, h1, loop2, b2)


# speedup vs baseline: 1.1036x; 1.0351x over previous
import functools

import jax
import jax.numpy as jnp
from jax import lax
from jax.experimental import pallas as pl
from jax.experimental.pallas import tpu as pltpu
from jax.experimental.pallas import tpu_sc as plsc

_NC = 2
_NS = 16
_NW = _NC * _NS
_CH = 128
_NSEG = 2
_LANES = 16


def _sc_gather_scatter(n_nodes, n_hid, e_pad, hrel, gidx, dst, nrm):
    epw = e_pad // _NW
    nchunk = epw // _CH
    n_pad = ((n_nodes + _NS * 8 - 1) // (_NS * 8)) * (_NS * 8)
    rpt = n_pad // _NS
    nvec = n_hid // _LANES

    mesh = plsc.VectorSubcoreMesh(core_axis_name="c", subcore_axis_name="s",
                                  num_cores=_NC, num_subcores=_NS)

    nseg = _NSEG
    cps = nchunk // nseg

    @functools.partial(
        pl.kernel,
        out_type=jax.ShapeDtypeStruct((_NC * n_pad, n_hid), jnp.float32),
        mesh=mesh,
        scratch_types=[
            pltpu.VMEM((cps, _CH), jnp.int32),
            pltpu.VMEM((cps, _CH), jnp.int32),
            pltpu.VMEM((cps * _CH,), jnp.float32),
            pltpu.VMEM((_CH, n_hid), jnp.float32),
            pltpu.VMEM((_CH, n_hid), jnp.float32),
            pltpu.VMEM_SHARED((n_pad, n_hid), jnp.float32),
            pltpu.SemaphoreType.DMA,
            pltpu.SemaphoreType.DMA,
        ],
    )
    def k(hrel_hbm, gidx_hbm, dst_hbm, nrm_hbm, out_hbm,
          idx_v, dst_v, nrm_v, rows0, rows1, agg_sh, sem0, sem1):
        cid = lax.axis_index("c")
        sid = lax.axis_index("s")
        wid = cid * _NS + sid

        zeros16 = jnp.zeros((_LANES,), jnp.float32)

        def zero_row(i, carry):
            for j in range(nvec):
                rows0[i, pl.ds(j * _LANES, _LANES)] = zeros16
            return carry

        lax.fori_loop(0, _CH, zero_row, 0)
        row0 = sid * rpt
        done = 0
        while done < rpt:
            sz = min(_CH, rpt - done)
            pltpu.sync_copy(rows0.at[pl.ds(0, sz)],
                            agg_sh.at[pl.ds(row0 + done, sz)])
            done += sz
        plsc.subcore_barrier()

        def scale(t, buf):
            def group(g, c2):
                norms = nrm_v[pl.ds(t * _CH + g * _LANES, _LANES)]
                e0 = g * _LANES
                for lane in range(_LANES):
                    s = norms[lane]
                    for j in range(nvec):
                        sl = pl.ds(j * _LANES, _LANES)
                        buf[e0 + lane, sl] = buf[e0 + lane, sl] * s
                return c2

            lax.fori_loop(0, _CH // _LANES, group, 0)

        def seg(s2, carry):
            pltpu.sync_copy(gidx_hbm.at[wid, s2], idx_v)
            pltpu.sync_copy(dst_hbm.at[wid, s2], dst_v)
            pltpu.sync_copy(nrm_hbm.at[wid, s2], nrm_v)
            pltpu.async_copy(hrel_hbm.at[idx_v.at[0]], rows0, sem0)

            def pair(p, c2):
                t0 = 2 * p
                t1 = t0 + 1
                pltpu.async_copy(hrel_hbm.at[idx_v.at[t1]], rows1, sem1)
                pltpu.make_async_copy(hrel_hbm.at[idx_v.at[t0]], rows0,
                                      sem0).wait()
                scale(t0, rows0)
                pltpu.sync_copy(rows0, agg_sh.at[dst_v.at[t0]], add=True)

                @pl.when(t1 + 1 < cps)
                def _():
                    pltpu.async_copy(hrel_hbm.at[idx_v.at[t1 + 1]], rows0, sem0)

                pltpu.make_async_copy(hrel_hbm.at[idx_v.at[t1]], rows1,
                                      sem1).wait()
                scale(t1, rows1)
                pltpu.sync_copy(rows1, agg_sh.at[dst_v.at[t1]], add=True)
                return c2

            lax.fori_loop(0, cps // 2, pair, 0)
            return carry

        lax.fori_loop(0, nseg, seg, 0)

        plsc.subcore_barrier()
        done = 0
        while done < rpt:
            sz = min(_CH, rpt - done)
            pltpu.sync_copy(agg_sh.at[pl.ds(row0 + done, sz)],
                            out_hbm.at[pl.ds(cid * n_pad + row0 + done, sz)])
            done += sz

    out = k(hrel,
            gidx.reshape(_NW, nseg, cps, _CH),
            dst.reshape(_NW, nseg, cps, _CH),
            nrm.reshape(_NW, nseg, cps * _CH))
    return out.reshape(_NC, n_pad, n_hid)


def _rel_matmul(x, ws):
    n, d = x.shape
    r1, _, h = ws.shape
    bn = 2000
    nb = n // bn

    def body(x_ref, w_ref, o_ref):
        o_ref[...] = jnp.dot(x_ref[...], w_ref[0],
                             preferred_element_type=jnp.float32)

    return pl.pallas_call(
        body,
        grid=(nb, r1),
        in_specs=[pl.BlockSpec((bn, d), lambda i, r: (i, 0)),
                  pl.BlockSpec((1, d, h), lambda i, r: (r, 0, 0))],
        out_specs=pl.BlockSpec((bn, h), lambda i, r: (r * nb + i, 0)),
        out_shape=jax.ShapeDtypeStruct((r1 * n, h), jnp.float32),
    )(x, ws)


def _combine_matmul(parts, x, wloop, b, ws):
    n, d = x.shape
    r1, _, h = ws.shape
    bn = 2000
    nb = n // bn

    def body(p_ref, x_ref, wl_ref, b_ref, w_ref, o2_ref, h1_ref, acc):
        r = pl.program_id(1)

        @pl.when(r == 0)
        def _():
            hblk = (p_ref[0] + p_ref[1] + b_ref[0]
                    + jnp.dot(x_ref[...], wl_ref[...],
                              preferred_element_type=jnp.float32))
            hblk = jnp.maximum(hblk, 0.0)
            acc[...] = hblk
            h1_ref[...] = hblk

        @pl.when(r > 0)
        def _():
            o2_ref[...] = jnp.dot(acc[...], w_ref[0],
                                  preferred_element_type=jnp.float32)

    hrel2, h1 = pl.pallas_call(
        body,
        grid=(nb, r1 + 1),
        in_specs=[pl.BlockSpec((2, bn, d), lambda i, r: (0, i, 0)),
                  pl.BlockSpec((bn, d), lambda i, r: (i, 0)),
                  pl.BlockSpec((d, d), lambda i, r: (0, 0)),
                  pl.BlockSpec((1, d), lambda i, r: (0, 0)),
                  pl.BlockSpec((1, d, h), lambda i, r: (jnp.maximum(r - 1, 0),
                                                        0, 0))],
        out_specs=[
            pl.BlockSpec((bn, h),
                         lambda i, r: (jnp.maximum(r - 1, 0) * nb + i, 0)),
            pl.BlockSpec((bn, d), lambda i, r: (i, 0)),
        ],
        out_shape=[jax.ShapeDtypeStruct((r1 * n, h), jnp.float32),
                   jax.ShapeDtypeStruct((n, d), jnp.float32)],
        scratch_shapes=[pltpu.VMEM((bn, d), jnp.float32)],
    )(parts, x, wloop, b.reshape(1, -1), ws)
    return hrel2, h1


def _final_sum(parts, h1, wloop, b):
    n, d = h1.shape
    bn = 2000
    nb = n // bn

    def body(p_ref, h_ref, wl_ref, b_ref, o_ref):
        o_ref[...] = (p_ref[0] + p_ref[1] + b_ref[0]
                      + jnp.dot(h_ref[...], wl_ref[...],
                                preferred_element_type=jnp.float32))

    return pl.pallas_call(
        body,
        grid=(nb,),
        in_specs=[pl.BlockSpec((2, bn, d), lambda i: (0, i, 0)),
                  pl.BlockSpec((bn, d), lambda i: (i, 0)),
                  pl.BlockSpec((d, d), lambda i: (0, 0)),
                  pl.BlockSpec((1, d), lambda i: (0, 0))],
        out_specs=pl.BlockSpec((bn, d), lambda i: (i, 0)),
        out_shape=jax.ShapeDtypeStruct((n, d), jnp.float32),
    )(parts, h1, wloop, b.reshape(1, -1))


def kernel(features, edge_index, etypes, norm, W1, loop1, b1, W2, loop2, b2):
    n, _ = features.shape
    e = etypes.shape[0]
    src = edge_index[0].astype(jnp.int32)
    dstv = edge_index[1].astype(jnp.int32)
    et = etypes.astype(jnp.int32)
    gidx = et * n + src
    nrm = norm[:, 0]

    granule = _NW * _CH * _NSEG
    e_pad = ((e + granule - 1) // granule) * granule
    pad = e_pad - e
    if pad:
        spread = jnp.arange(pad, dtype=jnp.int32) % n
        gidx = jnp.concatenate([gidx, spread])
        dstv = jnp.concatenate([dstv, spread])
        nrm = jnp.concatenate([nrm, jnp.zeros((pad,), jnp.float32)])

    hrel1 = _rel_matmul(features, W1)
    p1 = _sc_gather_scatter(n, W1.shape[2], e_pad, hrel1, gidx, dstv, nrm)
    hrel2, h1 = _combine_matmul(p1, features, loop1, b1, W2)
    p2 = _sc_gather_scatter(n, W2.shape[2], e_pad, hrel2, gidx, dstv, nrm)
    return _final_sum(p2, h1, loop2, b2)
